# baseline (device time: 11535 ns/iter reference)
import jax
import jax.numpy as jnp
from jax import lax
from jax.experimental import pallas as pl
from jax.experimental.pallas import tpu as pltpu

N_STRIPES = 4


def kernel(x, W, labels):
    t_tokens, d = x.shape
    d2, v_local = W.shape
    assert d == d2
    tv = v_local // N_STRIPES

    def body(x_hbm, w_hbm, lab_hbm, out_ref,
             x_vmem, w_vmem, lab_vmem, send_ref, recv_ref,
             wsems, xsem, lsem, send_sem, recv_sem):
        my_x = lax.axis_index("x")
        my_y = lax.axis_index("y")
        peer = (my_x, 1 - my_y)

        cx = pltpu.make_async_copy(x_hbm, x_vmem, xsem)
        cx.start()
        cl = pltpu.make_async_copy(lab_hbm, lab_vmem, lsem)
        cl.start()
        wcopies = []
        for k in range(N_STRIPES):
            c = pltpu.make_async_copy(
                w_hbm.at[:, pl.ds(k * tv, tv)],
                w_vmem.at[:, pl.ds(k * tv, tv)],
                wsems.at[k],
            )
            c.start()
            wcopies.append(c)

        barrier_sem = pltpu.get_barrier_semaphore()
        pl.semaphore_signal(barrier_sem, inc=1, device_id=peer,
                            device_id_type=pl.DeviceIdType.MESH)
        pl.semaphore_wait(barrier_sem, 1)

        cx.wait()
        cl.wait()
        xv = x_vmem[...]
        base = lab_vmem[0, :] - my_y * v_local

        m = jnp.full((t_tokens,), -1e30, jnp.float32)
        s = jnp.zeros((t_tokens,), jnp.float32)
        t = jnp.zeros((t_tokens,), jnp.float32)
        col = lax.broadcasted_iota(jnp.int32, (t_tokens, tv), 1)
        for k in range(N_STRIPES):
            wcopies[k].wait()
            logits = jnp.dot(xv, w_vmem[:, k * tv:(k + 1) * tv],
                             preferred_element_type=jnp.float32)
            lm = jnp.max(logits, axis=1)
            mn = jnp.maximum(m, lm)
            s = s * jnp.exp(m - mn) + jnp.sum(
                jnp.exp(logits - mn[:, None]), axis=1)
            m = mn
            tk = base - k * tv
            t = t + jnp.sum(jnp.where(col == tk[:, None], logits, 0.0),
                            axis=1)

        send_ref[0, :] = m
        send_ref[1, :] = s
        send_ref[2, :] = t

        rdma = pltpu.make_async_remote_copy(
            src_ref=send_ref,
            dst_ref=recv_ref,
            send_sem=send_sem,
            recv_sem=recv_sem,
            device_id=peer,
            device_id_type=pl.DeviceIdType.MESH,
        )
        rdma.start()
        rdma.wait()

        mb = recv_ref[0, :]
        sb = recv_ref[1, :]
        tb = recv_ref[2, :]
        mg = jnp.maximum(m, mb)
        sg = s * jnp.exp(m - mg) + sb * jnp.exp(mb - mg)
        out_ref[...] = mg + jnp.log(sg) - (t + tb)

    x = pltpu.with_memory_space_constraint(x, pltpu.MemorySpace.HBM)
    W = pltpu.with_memory_space_constraint(W, pltpu.MemorySpace.HBM)
    labels = pltpu.with_memory_space_constraint(
        labels.reshape(1, t_tokens), pltpu.MemorySpace.HBM)
    return pl.pallas_call(
        body,
        out_shape=jax.ShapeDtypeStruct((t_tokens,), jnp.float32),
        in_specs=[pl.BlockSpec(memory_space=pltpu.MemorySpace.HBM)] * 3,
        out_specs=pl.BlockSpec(memory_space=pltpu.VMEM),
        scratch_shapes=[
            pltpu.VMEM((t_tokens, d), jnp.float32),
            pltpu.VMEM((d, v_local), jnp.float32),
            pltpu.VMEM((1, t_tokens), jnp.int32),
            pltpu.VMEM((3, t_tokens), jnp.float32),
            pltpu.VMEM((3, t_tokens), jnp.float32),
            pltpu.SemaphoreType.DMA((N_STRIPES,)),
            pltpu.SemaphoreType.DMA,
            pltpu.SemaphoreType.DMA,
            pltpu.SemaphoreType.DMA,
            pltpu.SemaphoreType.DMA,
        ],
        compiler_params=pltpu.CompilerParams(collective_id=0),
    )(x, W, labels)


# device time: 10383 ns/iter; 1.1110x vs baseline; 1.1110x over previous
import jax
import jax.numpy as jnp
from jax import lax
from jax.experimental import pallas as pl
from jax.experimental.pallas import tpu as pltpu

N_STRIPES = 4


def kernel(x, W, labels):
    t_tokens, d = x.shape
    d2, v_local = W.shape
    assert d == d2
    tv = v_local // N_STRIPES

    def body(x_hbm, w_hbm, lab_hbm, out_ref,
             x_vmem, w_vmem, lab_vmem, send_ref, recv_ref,
             wsems, xsem, lsem, send_sem, recv_sem):
        my_x = lax.axis_index("x")
        my_y = lax.axis_index("y")
        peer = (my_x, 1 - my_y)

        barrier_sem = pltpu.get_barrier_semaphore()
        pl.semaphore_signal(barrier_sem, inc=1, device_id=peer,
                            device_id_type=pl.DeviceIdType.MESH)

        cx = pltpu.make_async_copy(x_hbm, x_vmem, xsem)
        cx.start()
        cl = pltpu.make_async_copy(lab_hbm, lab_vmem, lsem)
        cl.start()

        def w_copy(k):
            return pltpu.make_async_copy(
                w_hbm.at[:, pl.ds(k * tv, tv)],
                w_vmem.at[:, pl.ds(k * tv, tv)],
                wsems.at[k],
            )

        wcopies = [w_copy(k) for k in range(N_STRIPES)]
        wcopies[0].start()
        if N_STRIPES > 1:
            wcopies[1].start()

        cx.wait()
        cl.wait()
        xv = x_vmem[...]
        base = lab_vmem[0, :] - my_y * v_local

        m = jnp.full((t_tokens,), -1e30, jnp.float32)
        s = jnp.zeros((t_tokens,), jnp.float32)
        t = jnp.zeros((t_tokens,), jnp.float32)
        col = lax.broadcasted_iota(jnp.int32, (t_tokens, tv), 1)
        for k in range(N_STRIPES):
            wcopies[k].wait()
            if k + 2 < N_STRIPES:
                wcopies[k + 2].start()
            logits = jnp.dot(xv, w_vmem[:, k * tv:(k + 1) * tv],
                             preferred_element_type=jnp.float32)
            lm = jnp.max(logits, axis=1)
            mn = jnp.maximum(m, lm)
            s = s * jnp.exp(m - mn) + jnp.sum(
                jnp.exp(logits - mn[:, None]), axis=1)
            m = mn
            tk = base - k * tv
            t = t + jnp.sum(jnp.where(col == tk[:, None], logits, 0.0),
                            axis=1)

        send_ref[0, :] = m
        send_ref[1, :] = s
        send_ref[2, :] = t

        pl.semaphore_wait(barrier_sem, 1)
        rdma = pltpu.make_async_remote_copy(
            src_ref=send_ref,
            dst_ref=recv_ref,
            send_sem=send_sem,
            recv_sem=recv_sem,
            device_id=peer,
            device_id_type=pl.DeviceIdType.MESH,
        )
        rdma.start()
        rdma.wait_recv()

        mb = recv_ref[0, :]
        sb = recv_ref[1, :]
        tb = recv_ref[2, :]
        mg = jnp.maximum(m, mb)
        sg = s * jnp.exp(m - mg) + sb * jnp.exp(mb - mg)
        out_ref[...] = mg + jnp.log(sg) - (t + tb)
        rdma.wait_send()

    x = pltpu.with_memory_space_constraint(x, pltpu.MemorySpace.HBM)
    W = pltpu.with_memory_space_constraint(W, pltpu.MemorySpace.HBM)
    labels = pltpu.with_memory_space_constraint(
        labels.reshape(1, t_tokens), pltpu.MemorySpace.HBM)
    return pl.pallas_call(
        body,
        out_shape=jax.ShapeDtypeStruct((t_tokens,), jnp.float32),
        in_specs=[pl.BlockSpec(memory_space=pltpu.MemorySpace.HBM)] * 3,
        out_specs=pl.BlockSpec(memory_space=pltpu.VMEM),
        scratch_shapes=[
            pltpu.VMEM((t_tokens, d), jnp.float32),
            pltpu.VMEM((d, v_local), jnp.float32),
            pltpu.VMEM((1, t_tokens), jnp.int32),
            pltpu.VMEM((3, t_tokens), jnp.float32),
            pltpu.VMEM((3, t_tokens), jnp.float32),
            pltpu.SemaphoreType.DMA((N_STRIPES,)),
            pltpu.SemaphoreType.DMA,
            pltpu.SemaphoreType.DMA,
            pltpu.SemaphoreType.DMA,
            pltpu.SemaphoreType.DMA,
        ],
        compiler_params=pltpu.CompilerParams(collective_id=0),
    )(x, W, labels)


# device time: 10212 ns/iter; 1.1296x vs baseline; 1.0167x over previous
import jax
import jax.numpy as jnp
from jax import lax
from jax.experimental import pallas as pl
from jax.experimental.pallas import tpu as pltpu

N_STRIPES = 2


def kernel(x, W, labels):
    t_tokens, d = x.shape
    d2, v_local = W.shape
    assert d == d2
    tv = v_local // N_STRIPES

    def body(x_hbm, w_hbm, lab_hbm, out_ref,
             x_vmem, w_vmem, lab_vmem, send_ref, recv_ref,
             wsems, xsem, lsem, send_sem, recv_sem):
        my_x = lax.axis_index("x")
        my_y = lax.axis_index("y")
        peer = (my_x, 1 - my_y)

        barrier_sem = pltpu.get_barrier_semaphore()
        pl.semaphore_signal(barrier_sem, inc=1, device_id=peer,
                            device_id_type=pl.DeviceIdType.MESH)

        cx = pltpu.make_async_copy(x_hbm, x_vmem, xsem)
        cx.start()
        cl = pltpu.make_async_copy(lab_hbm, lab_vmem, lsem)
        cl.start()

        def w_copy(k):
            return pltpu.make_async_copy(
                w_hbm.at[:, pl.ds(k * tv, tv)],
                w_vmem.at[:, pl.ds(k * tv, tv)],
                wsems.at[k],
            )

        wcopies = [w_copy(k) for k in range(N_STRIPES)]
        wcopies[0].start()
        if N_STRIPES > 1:
            wcopies[1].start()

        cx.wait()
        cl.wait()
        xv = x_vmem[...]
        base = lab_vmem[0, :] - my_y * v_local

        m = jnp.full((t_tokens,), -1e30, jnp.float32)
        s = jnp.zeros((t_tokens,), jnp.float32)
        t = jnp.zeros((t_tokens,), jnp.float32)
        col = lax.broadcasted_iota(jnp.int32, (t_tokens, tv), 1)
        for k in range(N_STRIPES):
            wcopies[k].wait()
            if k + 2 < N_STRIPES:
                wcopies[k + 2].start()
            logits = jnp.dot(xv, w_vmem[:, k * tv:(k + 1) * tv],
                             preferred_element_type=jnp.float32)
            lm = jnp.max(logits, axis=1)
            mn = jnp.maximum(m, lm)
            s = s * jnp.exp(m - mn) + jnp.sum(
                jnp.exp(logits - mn[:, None]), axis=1)
            m = mn
            tk = base - k * tv
            t = t + jnp.sum(jnp.where(col == tk[:, None], logits, 0.0),
                            axis=1)

        send_ref[0, :] = m
        send_ref[1, :] = s
        send_ref[2, :] = t

        pl.semaphore_wait(barrier_sem, 1)
        rdma = pltpu.make_async_remote_copy(
            src_ref=send_ref,
            dst_ref=recv_ref,
            send_sem=send_sem,
            recv_sem=recv_sem,
            device_id=peer,
            device_id_type=pl.DeviceIdType.MESH,
        )
        rdma.start()
        rdma.wait_recv()

        mb = recv_ref[0, :]
        sb = recv_ref[1, :]
        tb = recv_ref[2, :]
        mg = jnp.maximum(m, mb)
        sg = s * jnp.exp(m - mg) + sb * jnp.exp(mb - mg)
        out_ref[...] = mg + jnp.log(sg) - (t + tb)
        rdma.wait_send()

    x = pltpu.with_memory_space_constraint(x, pltpu.MemorySpace.HBM)
    W = pltpu.with_memory_space_constraint(W, pltpu.MemorySpace.HBM)
    labels = pltpu.with_memory_space_constraint(
        labels.reshape(1, t_tokens), pltpu.MemorySpace.HBM)
    return pl.pallas_call(
        body,
        out_shape=jax.ShapeDtypeStruct((t_tokens,), jnp.float32),
        in_specs=[pl.BlockSpec(memory_space=pltpu.MemorySpace.HBM)] * 3,
        out_specs=pl.BlockSpec(memory_space=pltpu.VMEM),
        scratch_shapes=[
            pltpu.VMEM((t_tokens, d), jnp.float32),
            pltpu.VMEM((d, v_local), jnp.float32),
            pltpu.VMEM((1, t_tokens), jnp.int32),
            pltpu.VMEM((3, t_tokens), jnp.float32),
            pltpu.VMEM((3, t_tokens), jnp.float32),
            pltpu.SemaphoreType.DMA((N_STRIPES,)),
            pltpu.SemaphoreType.DMA,
            pltpu.SemaphoreType.DMA,
            pltpu.SemaphoreType.DMA,
            pltpu.SemaphoreType.DMA,
        ],
        compiler_params=pltpu.CompilerParams(collective_id=0),
    )(x, W, labels)
